# baseline (speedup 1.0000x reference)
import functools

import jax
import jax.numpy as jnp
from jax import lax
from jax.experimental import pallas as pl
from jax.experimental.pallas import tpu as pltpu
from jax.experimental.pallas import tpu_sc as plsc

_DIL = 2
_NEG_INF = float("-inf")

_SC_INFO = plsc.get_sparse_core_info()
_NC = _SC_INFO.num_cores
_NS = _SC_INFO.num_subcores
_NW = _NC * _NS
_CHUNK = 128


def _score_body(x_ref, mem_ref, memT_ref, gx_ref, idx_ref, w1_ref):
    P = x_ref.shape[1]
    M = mem_ref.shape[0]
    x = x_ref[0]
    mem = mem_ref[...]
    memT = memT_ref[...]

    ig = jnp.mean(x, axis=0, keepdims=True)
    sg = jnp.dot(ig, memT, preferred_element_type=jnp.float32)
    sg = sg - jnp.max(sg, axis=1, keepdims=True)
    eg = jnp.exp(sg)
    smg = eg / jnp.sum(eg, axis=1, keepdims=True)
    mr = jnp.dot(smg, mem, preferred_element_type=jnp.float32) + ig
    gate = 1.0 / (1.0 + jnp.exp(-mr))
    gx_ref[0] = x * gate

    S = jnp.dot(x, memT, preferred_element_type=jnp.float32)
    col = lax.broadcasted_iota(jnp.int32, (P, M), 1)
    v1 = jnp.max(S, axis=1, keepdims=True)
    i1 = jnp.min(jnp.where(S == v1, col, M), axis=1, keepdims=True)
    S2 = jnp.where(col == i1, _NEG_INF, S)
    v2 = jnp.max(S2, axis=1, keepdims=True)
    i2 = jnp.min(jnp.where(S2 == v2, col, M), axis=1, keepdims=True)
    e2 = jnp.exp(v2 - v1)
    idx_ref[0, 0] = i1
    idx_ref[1, 0] = i2
    w1_ref[0] = 1.0 / (1.0 + e2)


def _make_sc_gather(n_idx, D):
    rows_per_w = n_idx // _NW
    chunks = rows_per_w // _CHUNK
    mesh = plsc.VectorSubcoreMesh(core_axis_name="c", subcore_axis_name="s")

    @functools.partial(
        pl.kernel, mesh=mesh,
        out_type=jax.ShapeDtypeStruct((n_idx, D), jnp.float32),
        scratch_types=[
            pltpu.VMEM((chunks, _CHUNK), jnp.int32),
            pltpu.VMEM((_CHUNK, D), jnp.float32),
            pltpu.SemaphoreType.DMA,
        ],
    )
    def gather_k(table_hbm, idx_hbm, out_hbm, idx_v, rows_v, sem):
        wid = lax.axis_index("s") * _NC + lax.axis_index("c")
        base_chunk = wid * chunks
        pltpu.sync_copy(idx_hbm.at[pl.ds(base_chunk, chunks)], idx_v)
        for j in range(chunks):
            pltpu.async_copy(table_hbm.at[idx_v.at[j]], rows_v, sem).wait()
            pltpu.sync_copy(
                rows_v, out_hbm.at[pl.ds((base_chunk + j) * _CHUNK, _CHUNK)])

    return gather_k


def _fuse_body(H, W, r1_ref, r2_ref, w1_ref, gx_ref, wfa_ref, wfb_ref,
               bf_ref, taps_ref, bdw_ref, out_ref):
    C = gx_ref.shape[2]
    a1 = w1_ref[0]
    mf = a1 * r1_ref[0] + (1.0 - a1) * r2_ref[0]

    Y = (jnp.dot(gx_ref[0], wfa_ref[...], preferred_element_type=jnp.float32)
         + jnp.dot(mf, wfb_ref[...], preferred_element_type=jnp.float32)
         + bf_ref[...])
    Y = jnp.where(Y > 0, Y, 0.2 * Y)

    Yh = Y.reshape(H, W, C)

    def shift(a, axis, d):
        if d == 0:
            return a
        zshape = list(a.shape)
        zshape[axis] = abs(d)
        z = jnp.zeros(zshape, a.dtype)
        n = a.shape[axis]
        if d > 0:
            body = lax.slice_in_dim(a, d, n, axis=axis)
            return jnp.concatenate([body, z], axis=axis)
        body = lax.slice_in_dim(a, 0, n + d, axis=axis)
        return jnp.concatenate([z, body], axis=axis)

    acc = jnp.zeros((H, W, C), jnp.float32)
    k = 0
    for kh in range(3):
        for kw in range(3):
            dh = (kh - 1) * _DIL
            dw = (kw - 1) * _DIL
            win = shift(shift(Yh, 0, dh), 1, dw)
            acc = acc + win * taps_ref[k, :][None, None, :]
            k += 1
    acc = acc + bdw_ref[0, :][None, None, :]
    out_ref[0] = jnp.where(acc > 0, acc, 0.2 * acc)


def kernel(image_feature, memory, W_fuse, b_fuse, W_dw, b_dw):
    B, C, H, W = image_feature.shape
    M = memory.shape[0]
    P = H * W
    x_pc = image_feature.reshape(B, C, P).transpose(0, 2, 1)
    memT = memory.T
    wfa = W_fuse[:, :C].T
    wfb = W_fuse[:, C:].T
    taps = W_dw[:, 0, :, :].reshape(C, 9).T
    bf = b_fuse.reshape(1, C)
    bdw = b_dw.reshape(1, C)

    gx, idx, w1 = pl.pallas_call(
        _score_body,
        grid=(B,),
        in_specs=[
            pl.BlockSpec((1, P, C), lambda b: (b, 0, 0)),
            pl.BlockSpec((M, C), lambda b: (0, 0)),
            pl.BlockSpec((C, M), lambda b: (0, 0)),
        ],
        out_specs=[
            pl.BlockSpec((1, P, C), lambda b: (b, 0, 0)),
            pl.BlockSpec((2, 1, P, 1), lambda b: (0, b, 0, 0)),
            pl.BlockSpec((1, P, 1), lambda b: (b, 0, 0)),
        ],
        out_shape=[
            jax.ShapeDtypeStruct((B, P, C), jnp.float32),
            jax.ShapeDtypeStruct((2, B, P, 1), jnp.int32),
            jax.ShapeDtypeStruct((B, P, 1), jnp.float32),
        ],
    )(x_pc, memory, memT)

    n_idx = 2 * B * P
    idx_flat = idx.reshape(n_idx // _CHUNK, _CHUNK)
    gathered = _make_sc_gather(n_idx, C)(memory, idx_flat)
    r1 = gathered[:B * P].reshape(B, P, C)
    r2 = gathered[B * P:].reshape(B, P, C)

    out = pl.pallas_call(
        functools.partial(_fuse_body, H, W),
        grid=(B,),
        in_specs=[
            pl.BlockSpec((1, P, C), lambda b: (b, 0, 0)),
            pl.BlockSpec((1, P, C), lambda b: (b, 0, 0)),
            pl.BlockSpec((1, P, 1), lambda b: (b, 0, 0)),
            pl.BlockSpec((1, P, C), lambda b: (b, 0, 0)),
            pl.BlockSpec((C, C), lambda b: (0, 0)),
            pl.BlockSpec((C, C), lambda b: (0, 0)),
            pl.BlockSpec((1, C), lambda b: (0, 0)),
            pl.BlockSpec((9, C), lambda b: (0, 0)),
            pl.BlockSpec((1, C), lambda b: (0, 0)),
        ],
        out_specs=pl.BlockSpec((1, H, W, C), lambda b: (b, 0, 0, 0)),
        out_shape=jax.ShapeDtypeStruct((B, H, W, C), jnp.float32),
    )(r1, r2, w1, gx, wfa, wfb, bf, taps, bdw)
    return out.transpose(0, 3, 1, 2)
